# baseline (speedup 1.0000x reference)
import functools

import jax
import jax.numpy as jnp
from jax import lax
from jax.experimental import pallas as pl
from jax.experimental.pallas import tpu as pltpu
from jax.experimental.pallas import tpu_sc as plsc

_BATCH = 20
_SEQ = 16384
_DIM = 64

_INFO = plsc.get_sparse_core_info()
_NC = _INFO.num_cores
_NS = _INFO.num_subcores
_NW = _NC * _NS

_N = _BATCH * _SEQ
_PER_W = _N // _NW
_G = 128
_H = _G // 2
_NG = _PER_W // _G
_NBUF = 8


_TC = 256
_TNB = 1000000 // _TC
_TTAIL = 1000000 - _TNB * _TC


def _tbody(tblt, tail_hbm, lin_hbm, buf, buft):
    c = lax.axis_index("c")
    s = lax.axis_index("s")
    wid = s * _NC + c
    iota = lax.iota(jnp.int32, 16)

    def _transpose(c0, ncols):
        def _pair(p, carry2):
            for half in range(2):
                rvec = jnp.zeros((16,), jnp.int32) + (2 * p + half)
                for d0 in range(0, _DIM, 16):
                    v = plsc.load_gather(buf, [d0 + iota, rvec])
                    buft[p, pl.ds(half * _DIM + d0, 16)] = v
            return carry2

        lax.fori_loop(0, ncols // 2, _pair, 0)

    def _block(k, carry):
        blk = wid + k * _NW

        @pl.when(blk < _TNB)
        def _():
            c0 = pl.multiple_of(blk * _TC, _TC)
            pltpu.sync_copy(tblt.at[:, pl.ds(c0, _TC)], buf)
            _transpose(c0, _TC)
            pltpu.sync_copy(
                buft, lin_hbm.at[pl.ds(pl.multiple_of(blk * (_TC // 2), _TC // 2),
                                       _TC // 2)])

        return carry

    lax.fori_loop(0, (_TNB + _NW - 1) // _NW, _block, 0)

    @pl.when(wid == _NW - 1)
    def _():
        pltpu.sync_copy(tail_hbm,
                        lin_hbm.at[pl.ds((_TNB * _TC) // 2, _TTAIL // 2)])


@jax.jit
def _linearize(table, tail):
    mesh = plsc.VectorSubcoreMesh(core_axis_name="c", subcore_axis_name="s")
    run = pl.kernel(
        _tbody,
        out_type=jax.ShapeDtypeStruct((500000, 128), jnp.float32),
        mesh=mesh,
        scratch_types=[
            pltpu.VMEM((_DIM, _TC), jnp.float32),
            pltpu.VMEM((_TC // 2, 128), jnp.float32),
        ],
        name="linearize_table",
        compiler_params=pltpu.CompilerParams(use_tc_tiling_on_sc=True,
                                             needs_layout_passes=False),
    )
    return run(table.T, tail)


def _body(tbl, idxe_hbm, idxo_hbm, out_hbm, idxe_v, idxo_v, rowse, rowso, gsems):
    c = lax.axis_index("c")
    s = lax.axis_index("s")
    wid = s * _NC + c
    base = wid * (_PER_W // 2)

    pltpu.sync_copy(idxe_hbm.at[wid], idxe_v)
    pltpu.sync_copy(idxo_hbm.at[wid], idxo_v)

    def _fire(j, b):
        pltpu.async_copy(tbl.at[idxe_v.at[j]], rowse.at[b], gsems.at[b])
        pltpu.async_copy(tbl.at[idxo_v.at[j]], rowso.at[b], gsems.at[b])

    def _wait(j, b):
        pltpu.make_async_copy(tbl.at[idxe_v.at[j]], rowse.at[b],
                              gsems.at[b]).wait()
        pltpu.make_async_copy(tbl.at[idxo_v.at[j]], rowso.at[b],
                              gsems.at[b]).wait()

    def _store(j, b):
        r0 = base + j * _H
        pltpu.sync_copy(rowse.at[b], out_hbm.at[pl.ds(r0, _H), pl.ds(0, _DIM)])
        pltpu.sync_copy(rowso.at[b], out_hbm.at[pl.ds(r0, _H), pl.ds(_DIM, _DIM)])

    for b in range(_NBUF):
        _fire(b, b)

    def _group(g, carry):
        for b in range(_NBUF):
            j = g * _NBUF + b
            _wait(j, b)
            _store(j, b)
            _fire(j + _NBUF, b)
        return carry

    lax.fori_loop(0, _NG // _NBUF - 1, _group, 0)

    for b in range(_NBUF):
        j = _NG - _NBUF + b
        _wait(j, b)
        _store(j, b)


@jax.jit
def _gather(table, idxe, idxo):
    mesh = plsc.VectorSubcoreMesh(core_axis_name="c", subcore_axis_name="s")
    run = pl.kernel(
        _body,
        out_type=jax.ShapeDtypeStruct((_N // 2, 2 * _DIM), jnp.float32),
        mesh=mesh,
        scratch_types=[
            pltpu.VMEM((_NG, _H), jnp.int32),
            pltpu.VMEM((_NG, _H), jnp.int32),
            pltpu.VMEM((_NBUF, _H, _DIM), jnp.float32),
            pltpu.VMEM((_NBUF, _H, _DIM), jnp.float32),
            pltpu.SemaphoreType.DMA((_NBUF,)),
        ],
        compiler_params=pltpu.CompilerParams(use_tc_tiling_on_sc=False),
    )
    return run(table, idxe, idxo)


def kernel(inputs, embedding_weight):
    idx = inputs.reshape(_NW, _NG, _H, 2).astype(jnp.int32)
    tail = embedding_weight[_TNB * _TC:].reshape(_TTAIL // 2, 128)
    lin = _linearize(embedding_weight, tail).reshape(1000000, 64)
    rows = _gather(lin, idx[..., 0], idx[..., 1])
    return rows.reshape(_BATCH, -1)


# speedup vs baseline: 1.0218x; 1.0218x over previous
import functools

import jax
import jax.numpy as jnp
from jax import lax
from jax.experimental import pallas as pl
from jax.experimental.pallas import tpu as pltpu
from jax.experimental.pallas import tpu_sc as plsc

_BATCH = 20
_SEQ = 16384
_DIM = 64

_INFO = plsc.get_sparse_core_info()
_NC = _INFO.num_cores
_NS = _INFO.num_subcores
_NW = _NC * _NS

_N = _BATCH * _SEQ
_PER_W = _N // _NW
_G = 128
_H = _G // 2
_NG = _PER_W // _G
_NBUF = 8


_TC = 512
_TNB = 1000000 // _TC
_TTAIL = 1000000 - _TNB * _TC
_TU = 8


def _tbody(tblt, tail_hbm, lin_hbm, buf, buft):
    c = lax.axis_index("c")
    s = lax.axis_index("s")
    wid = s * _NC + c
    iota = lax.iota(jnp.int32, 16)

    def _transpose(ncols):
        def _grp(g, carry2):
            p0 = g * _TU
            for dp in range(_TU):
                p = p0 + dp
                for half in range(2):
                    rvec = jnp.zeros((16,), jnp.int32) + (2 * p + half)
                    for d0 in range(0, _DIM, 16):
                        v = plsc.load_gather(buf, [d0 + iota, rvec])
                        buft[p, pl.ds(half * _DIM + d0, 16)] = v
            return carry2

        lax.fori_loop(0, ncols // (2 * _TU), _grp, 0)

    def _block(k, carry):
        blk = wid + k * _NW

        @pl.when(blk < _TNB)
        def _():
            c0 = pl.multiple_of(blk * _TC, _TC)
            pltpu.sync_copy(tblt.at[:, pl.ds(c0, _TC)], buf)
            _transpose(_TC)
            pltpu.sync_copy(
                buft, lin_hbm.at[pl.ds(pl.multiple_of(blk * (_TC // 2), _TC // 2),
                                       _TC // 2)])

        return carry

    lax.fori_loop(0, (_TNB + _NW - 1) // _NW, _block, 0)

    @pl.when(wid == _NW - 1)
    def _():
        pltpu.sync_copy(tail_hbm,
                        lin_hbm.at[pl.ds((_TNB * _TC) // 2, _TTAIL // 2)])


@jax.jit
def _linearize(table, tail):
    mesh = plsc.VectorSubcoreMesh(core_axis_name="c", subcore_axis_name="s")
    run = pl.kernel(
        _tbody,
        out_type=jax.ShapeDtypeStruct((500000, 128), jnp.float32),
        mesh=mesh,
        scratch_types=[
            pltpu.VMEM((_DIM, _TC), jnp.float32),
            pltpu.VMEM((_TC // 2, 128), jnp.float32),
        ],
        name="linearize_table",
        compiler_params=pltpu.CompilerParams(use_tc_tiling_on_sc=True,
                                             needs_layout_passes=False),
    )
    return run(table.T, tail)


def _body(tbl, idxe_hbm, idxo_hbm, out_hbm, idxe_v, idxo_v, rowse, rowso, gsems):
    c = lax.axis_index("c")
    s = lax.axis_index("s")
    wid = s * _NC + c
    base = wid * (_PER_W // 2)

    pltpu.sync_copy(idxe_hbm.at[wid], idxe_v)
    pltpu.sync_copy(idxo_hbm.at[wid], idxo_v)

    def _fire(j, b):
        pltpu.async_copy(tbl.at[idxe_v.at[j]], rowse.at[b], gsems.at[b])
        pltpu.async_copy(tbl.at[idxo_v.at[j]], rowso.at[b], gsems.at[b])

    def _wait(j, b):
        pltpu.make_async_copy(tbl.at[idxe_v.at[j]], rowse.at[b],
                              gsems.at[b]).wait()
        pltpu.make_async_copy(tbl.at[idxo_v.at[j]], rowso.at[b],
                              gsems.at[b]).wait()

    def _store(j, b):
        r0 = base + j * _H
        pltpu.sync_copy(rowse.at[b], out_hbm.at[pl.ds(r0, _H), pl.ds(0, _DIM)])
        pltpu.sync_copy(rowso.at[b], out_hbm.at[pl.ds(r0, _H), pl.ds(_DIM, _DIM)])

    for b in range(_NBUF):
        _fire(b, b)

    def _group(g, carry):
        for b in range(_NBUF):
            j = g * _NBUF + b
            _wait(j, b)
            _store(j, b)
            _fire(j + _NBUF, b)
        return carry

    lax.fori_loop(0, _NG // _NBUF - 1, _group, 0)

    for b in range(_NBUF):
        j = _NG - _NBUF + b
        _wait(j, b)
        _store(j, b)


@jax.jit
def _gather(table, idxe, idxo):
    mesh = plsc.VectorSubcoreMesh(core_axis_name="c", subcore_axis_name="s")
    run = pl.kernel(
        _body,
        out_type=jax.ShapeDtypeStruct((_N // 2, 2 * _DIM), jnp.float32),
        mesh=mesh,
        scratch_types=[
            pltpu.VMEM((_NG, _H), jnp.int32),
            pltpu.VMEM((_NG, _H), jnp.int32),
            pltpu.VMEM((_NBUF, _H, _DIM), jnp.float32),
            pltpu.VMEM((_NBUF, _H, _DIM), jnp.float32),
            pltpu.SemaphoreType.DMA((_NBUF,)),
        ],
        compiler_params=pltpu.CompilerParams(use_tc_tiling_on_sc=False),
    )
    return run(table, idxe, idxo)


def kernel(inputs, embedding_weight):
    idx = inputs.reshape(_NW, _NG, _H, 2).astype(jnp.int32)
    tail = embedding_weight[_TNB * _TC:].reshape(_TTAIL // 2, 128)
    lin = _linearize(embedding_weight, tail).reshape(1000000, 64)
    rows = _gather(lin, idx[..., 0], idx[..., 1])
    return rows.reshape(_BATCH, -1)


# speedup vs baseline: 1.6310x; 1.5962x over previous
import functools

import jax
import jax.numpy as jnp
from jax import lax
from jax.experimental import pallas as pl
from jax.experimental.pallas import tpu as pltpu
from jax.experimental.pallas import tpu_sc as plsc

_BATCH = 20
_SEQ = 16384
_DIM = 64

_INFO = plsc.get_sparse_core_info()
_NC = _INFO.num_cores
_NS = _INFO.num_subcores
_NW = _NC * _NS

_N = _BATCH * _SEQ
_PER_W = _N // _NW
_G = 128
_H = _G // 2
_NG = _PER_W // _G
_NBUF = 8


_TC = 512
_TNB = 1000000 // _TC
_TTAIL = 1000000 - _TNB * _TC
_TU = 8


def _tbody(tblt, tail_hbm, lin_hbm, buf, buft):
    c = lax.axis_index("c")
    s = lax.axis_index("s")
    wid = s * _NC + c
    iota = lax.iota(jnp.int32, 16)

    def _transpose(ncols):
        @plsc.parallel_loop(0, ncols // 2, unroll=_TU)
        def _pair(p):
            for half in range(2):
                rvec = jnp.zeros((16,), jnp.int32) + (2 * p + half)
                for d0 in range(0, _DIM, 16):
                    v = plsc.load_gather(buf, [d0 + iota, rvec])
                    buft[p, pl.ds(half * _DIM + d0, 16)] = v

    def _block(k, carry):
        blk = wid + k * _NW

        @pl.when(blk < _TNB)
        def _():
            c0 = pl.multiple_of(blk * _TC, _TC)
            pltpu.sync_copy(tblt.at[:, pl.ds(c0, _TC)], buf)
            _transpose(_TC)
            pltpu.sync_copy(
                buft, lin_hbm.at[pl.ds(pl.multiple_of(blk * (_TC // 2), _TC // 2),
                                       _TC // 2)])

        return carry

    lax.fori_loop(0, (_TNB + _NW - 1) // _NW, _block, 0)

    @pl.when(wid == _NW - 1)
    def _():
        pltpu.sync_copy(tail_hbm,
                        lin_hbm.at[pl.ds((_TNB * _TC) // 2, _TTAIL // 2)])


@jax.jit
def _linearize(table, tail):
    mesh = plsc.VectorSubcoreMesh(core_axis_name="c", subcore_axis_name="s")
    run = pl.kernel(
        _tbody,
        out_type=jax.ShapeDtypeStruct((500000, 128), jnp.float32),
        mesh=mesh,
        scratch_types=[
            pltpu.VMEM((_DIM, _TC), jnp.float32),
            pltpu.VMEM((_TC // 2, 128), jnp.float32),
        ],
        name="linearize_table",
        compiler_params=pltpu.CompilerParams(use_tc_tiling_on_sc=True,
                                             needs_layout_passes=False),
    )
    return run(table.T, tail)


def _body(tbl, idxe_hbm, idxo_hbm, out_hbm, idxe_v, idxo_v, rowse, rowso, gsems):
    c = lax.axis_index("c")
    s = lax.axis_index("s")
    wid = s * _NC + c
    base = wid * (_PER_W // 2)

    pltpu.sync_copy(idxe_hbm.at[wid], idxe_v)
    pltpu.sync_copy(idxo_hbm.at[wid], idxo_v)

    def _fire(j, b):
        pltpu.async_copy(tbl.at[idxe_v.at[j]], rowse.at[b], gsems.at[b])
        pltpu.async_copy(tbl.at[idxo_v.at[j]], rowso.at[b], gsems.at[b])

    def _wait(j, b):
        pltpu.make_async_copy(tbl.at[idxe_v.at[j]], rowse.at[b],
                              gsems.at[b]).wait()
        pltpu.make_async_copy(tbl.at[idxo_v.at[j]], rowso.at[b],
                              gsems.at[b]).wait()

    def _store(j, b):
        r0 = base + j * _H
        pltpu.sync_copy(rowse.at[b], out_hbm.at[pl.ds(r0, _H), pl.ds(0, _DIM)])
        pltpu.sync_copy(rowso.at[b], out_hbm.at[pl.ds(r0, _H), pl.ds(_DIM, _DIM)])

    for b in range(_NBUF):
        _fire(b, b)

    def _group(g, carry):
        for b in range(_NBUF):
            j = g * _NBUF + b
            _wait(j, b)
            _store(j, b)
            _fire(j + _NBUF, b)
        return carry

    lax.fori_loop(0, _NG // _NBUF - 1, _group, 0)

    for b in range(_NBUF):
        j = _NG - _NBUF + b
        _wait(j, b)
        _store(j, b)


@jax.jit
def _gather(table, idxe, idxo):
    mesh = plsc.VectorSubcoreMesh(core_axis_name="c", subcore_axis_name="s")
    run = pl.kernel(
        _body,
        out_type=jax.ShapeDtypeStruct((_N // 2, 2 * _DIM), jnp.float32),
        mesh=mesh,
        scratch_types=[
            pltpu.VMEM((_NG, _H), jnp.int32),
            pltpu.VMEM((_NG, _H), jnp.int32),
            pltpu.VMEM((_NBUF, _H, _DIM), jnp.float32),
            pltpu.VMEM((_NBUF, _H, _DIM), jnp.float32),
            pltpu.SemaphoreType.DMA((_NBUF,)),
        ],
        compiler_params=pltpu.CompilerParams(use_tc_tiling_on_sc=False),
    )
    return run(table, idxe, idxo)


def kernel(inputs, embedding_weight):
    idx = inputs.reshape(_NW, _NG, _H, 2).astype(jnp.int32)
    tail = embedding_weight[_TNB * _TC:].reshape(_TTAIL // 2, 128)
    lin = _linearize(embedding_weight, tail).reshape(1000000, 64)
    rows = _gather(lin, idx[..., 0], idx[..., 1])
    return rows.reshape(_BATCH, -1)


# speedup vs baseline: 1.7769x; 1.0894x over previous
import functools

import jax
import jax.numpy as jnp
from jax import lax
from jax.experimental import pallas as pl
from jax.experimental.pallas import tpu as pltpu
from jax.experimental.pallas import tpu_sc as plsc

_BATCH = 20
_SEQ = 16384
_DIM = 64

_INFO = plsc.get_sparse_core_info()
_NC = _INFO.num_cores
_NS = _INFO.num_subcores
_NW = _NC * _NS

_N = _BATCH * _SEQ
_PER_W = _N // _NW
_G = 128
_H = _G // 2
_NG = _PER_W // _G
_NBUF = 8


_TC = 512
_TNB = 1000000 // _TC
_TTAIL = 1000000 - _TNB * _TC
_TU = 8


def _tbody(tblt, tail_hbm, lin_hbm, buf, buft):
    c = lax.axis_index("c")
    s = lax.axis_index("s")
    wid = s * _NC + c
    iota = lax.iota(jnp.int32, 16)

    def _transpose(ncols):
        @plsc.parallel_loop(0, ncols // 16, unroll=2)
        def _rtile(rt):
            r0 = rt * 16
            for d0 in range(0, _DIM, 16):
                dvec = d0 + iota
                for k in range(16):
                    rv = r0 + ((iota + k) & 15)
                    v = plsc.load_gather(buf, [dvec, rv])
                    p = lax.shift_right_logical(rv, 1)
                    q = lax.shift_left(rv & 1, 6) + dvec
                    plsc.store_scatter(buft, [p, q], v)

    def _block(k, carry):
        blk = wid + k * _NW

        @pl.when(blk < _TNB)
        def _():
            c0 = pl.multiple_of(blk * _TC, _TC)
            pltpu.sync_copy(tblt.at[:, pl.ds(c0, _TC)], buf)
            _transpose(_TC)
            pltpu.sync_copy(
                buft, lin_hbm.at[pl.ds(pl.multiple_of(blk * (_TC // 2), _TC // 2),
                                       _TC // 2)])

        return carry

    lax.fori_loop(0, (_TNB + _NW - 1) // _NW, _block, 0)

    @pl.when(wid == _NW - 1)
    def _():
        pltpu.sync_copy(tail_hbm,
                        lin_hbm.at[pl.ds((_TNB * _TC) // 2, _TTAIL // 2)])


@jax.jit
def _linearize(table, tail):
    mesh = plsc.VectorSubcoreMesh(core_axis_name="c", subcore_axis_name="s")
    run = pl.kernel(
        _tbody,
        out_type=jax.ShapeDtypeStruct((500000, 128), jnp.float32),
        mesh=mesh,
        scratch_types=[
            pltpu.VMEM((_DIM, _TC), jnp.float32),
            pltpu.VMEM((_TC // 2, 128), jnp.float32),
        ],
        name="linearize_table",
        compiler_params=pltpu.CompilerParams(use_tc_tiling_on_sc=True,
                                             needs_layout_passes=False),
    )
    return run(table.T, tail)


def _body(tbl, idxe_hbm, idxo_hbm, out_hbm, idxe_v, idxo_v, rowse, rowso, gsems):
    c = lax.axis_index("c")
    s = lax.axis_index("s")
    wid = s * _NC + c
    base = wid * (_PER_W // 2)

    pltpu.sync_copy(idxe_hbm.at[wid], idxe_v)
    pltpu.sync_copy(idxo_hbm.at[wid], idxo_v)

    def _fire(j, b):
        pltpu.async_copy(tbl.at[idxe_v.at[j]], rowse.at[b], gsems.at[b])
        pltpu.async_copy(tbl.at[idxo_v.at[j]], rowso.at[b], gsems.at[b])

    def _wait(j, b):
        pltpu.make_async_copy(tbl.at[idxe_v.at[j]], rowse.at[b],
                              gsems.at[b]).wait()
        pltpu.make_async_copy(tbl.at[idxo_v.at[j]], rowso.at[b],
                              gsems.at[b]).wait()

    def _store(j, b):
        r0 = base + j * _H
        pltpu.sync_copy(rowse.at[b], out_hbm.at[pl.ds(r0, _H), pl.ds(0, _DIM)])
        pltpu.sync_copy(rowso.at[b], out_hbm.at[pl.ds(r0, _H), pl.ds(_DIM, _DIM)])

    for b in range(_NBUF):
        _fire(b, b)

    def _group(g, carry):
        for b in range(_NBUF):
            j = g * _NBUF + b
            _wait(j, b)
            _store(j, b)
            _fire(j + _NBUF, b)
        return carry

    lax.fori_loop(0, _NG // _NBUF - 1, _group, 0)

    for b in range(_NBUF):
        j = _NG - _NBUF + b
        _wait(j, b)
        _store(j, b)


@jax.jit
def _gather(table, idxe, idxo):
    mesh = plsc.VectorSubcoreMesh(core_axis_name="c", subcore_axis_name="s")
    run = pl.kernel(
        _body,
        out_type=jax.ShapeDtypeStruct((_N // 2, 2 * _DIM), jnp.float32),
        mesh=mesh,
        scratch_types=[
            pltpu.VMEM((_NG, _H), jnp.int32),
            pltpu.VMEM((_NG, _H), jnp.int32),
            pltpu.VMEM((_NBUF, _H, _DIM), jnp.float32),
            pltpu.VMEM((_NBUF, _H, _DIM), jnp.float32),
            pltpu.SemaphoreType.DMA((_NBUF,)),
        ],
        compiler_params=pltpu.CompilerParams(use_tc_tiling_on_sc=False),
    )
    return run(table, idxe, idxo)


def kernel(inputs, embedding_weight):
    idx = inputs.reshape(_NW, _NG, _H, 2).astype(jnp.int32)
    tail = embedding_weight[_TNB * _TC:].reshape(_TTAIL // 2, 128)
    lin = _linearize(embedding_weight, tail).reshape(1000000, 64)
    rows = _gather(lin, idx[..., 0], idx[..., 1])
    return rows.reshape(_BATCH, -1)


# speedup vs baseline: 2.0222x; 1.1381x over previous
import functools

import jax
import jax.numpy as jnp
from jax import lax
from jax.experimental import pallas as pl
from jax.experimental.pallas import tpu as pltpu
from jax.experimental.pallas import tpu_sc as plsc

_BATCH = 20
_SEQ = 16384
_DIM = 64

_INFO = plsc.get_sparse_core_info()
_NC = _INFO.num_cores
_NS = _INFO.num_subcores
_NW = _NC * _NS

_N = _BATCH * _SEQ
_PER_W = _N // _NW
_G = 128
_H = _G // 2
_NG = _PER_W // _G
_NBUF = 8


def _body(tbl, idxe_hbm, idxo_hbm, out_hbm, idxe_v, idxo_v, rowse, rowso, gsems):
    c = lax.axis_index("c")
    s = lax.axis_index("s")
    wid = s * _NC + c
    base = wid * (_PER_W // 2)

    pltpu.sync_copy(idxe_hbm.at[wid], idxe_v)
    pltpu.sync_copy(idxo_hbm.at[wid], idxo_v)

    def _fire(j, b):
        pltpu.async_copy(tbl.at[idxe_v.at[j]], rowse.at[b], gsems.at[b])
        pltpu.async_copy(tbl.at[idxo_v.at[j]], rowso.at[b], gsems.at[b])

    def _wait(j, b):
        pltpu.make_async_copy(tbl.at[idxe_v.at[j]], rowse.at[b],
                              gsems.at[b]).wait()
        pltpu.make_async_copy(tbl.at[idxo_v.at[j]], rowso.at[b],
                              gsems.at[b]).wait()

    def _store(j, b):
        r0 = base + j * _H
        pltpu.sync_copy(rowse.at[b], out_hbm.at[pl.ds(r0, _H), pl.ds(0, _DIM)])
        pltpu.sync_copy(rowso.at[b], out_hbm.at[pl.ds(r0, _H), pl.ds(_DIM, _DIM)])

    for b in range(_NBUF):
        _fire(b, b)

    def _group(g, carry):
        for b in range(_NBUF):
            j = g * _NBUF + b
            _wait(j, b)
            _store(j, b)
            _fire(j + _NBUF, b)
        return carry

    lax.fori_loop(0, _NG // _NBUF - 1, _group, 0)

    for b in range(_NBUF):
        j = _NG - _NBUF + b
        _wait(j, b)
        _store(j, b)


@jax.jit
def _gather(table, idxe, idxo):
    mesh = plsc.VectorSubcoreMesh(core_axis_name="c", subcore_axis_name="s")
    run = pl.kernel(
        _body,
        out_type=jax.ShapeDtypeStruct((_N // 2, 2 * _DIM), jnp.float32),
        mesh=mesh,
        scratch_types=[
            pltpu.VMEM((_NG, _H), jnp.int32),
            pltpu.VMEM((_NG, _H), jnp.int32),
            pltpu.VMEM((_NBUF, _H, _DIM), jnp.float32),
            pltpu.VMEM((_NBUF, _H, _DIM), jnp.float32),
            pltpu.SemaphoreType.DMA((_NBUF,)),
        ],
        compiler_params=pltpu.CompilerParams(use_tc_tiling_on_sc=False),
    )
    return run(table, idxe, idxo)


def kernel(inputs, embedding_weight):
    idx = inputs.reshape(_NW, _NG, _H, 2).astype(jnp.int32)
    rows = _gather(embedding_weight, idx[..., 0], idx[..., 1])
    return rows.reshape(_BATCH, -1)
